# baseline (speedup 1.0000x reference)
import functools

import jax
import jax.numpy as jnp
from jax import lax
from jax.experimental import pallas as pl
from jax.experimental.pallas import tpu as pltpu
from jax.experimental.pallas import tpu_sc as plsc

B = 16384
K = 20
AGE_NUM = 8
GENDER_NUM = 3
OCC_NUM = 22
KIND_NUM = 19


_NC, _NS = 2, 16
_NW = _NC * _NS
_CH = B // _NW
_CR = _CH // 128


def _sc_body(ut_hbm, it_hbm, uid_hbm, iid_hbm, out_hbm,
             uidx_v, iidx_v, urow_v, irow_v, sum_v, sem):
    wid = lax.axis_index("s") * _NC + lax.axis_index("c")
    pltpu.sync_copy(uid_hbm.at[wid], uidx_v)
    pltpu.sync_copy(iid_hbm.at[wid], iidx_v)
    cps = []
    for j in range(_CR):
        cps.append(pltpu.async_copy(ut_hbm.at[uidx_v.at[j]], urow_v.at[j], sem))
        cps.append(pltpu.async_copy(it_hbm.at[iidx_v.at[j]], irow_v.at[j], sem))
    for cp in cps:
        cp.wait()
    for j in range(_CR):
        for i in range(8):
            s = pl.ds(i * 16, 16)
            sum_v[j, s] = urow_v[j, s] + irow_v[j, s]
    pltpu.sync_copy(sum_v, out_hbm.at[wid])


def _sc_ui_sum(user_table, item_table, uid, iid):
    mesh = plsc.VectorSubcoreMesh(core_axis_name="c", subcore_axis_name="s")
    k = pl.kernel(
        _sc_body,
        out_type=jax.ShapeDtypeStruct((_NW, _CR, 128), jnp.float32),
        mesh=mesh,
        scratch_types=[
            pltpu.VMEM((_CR, 128), jnp.int32),
            pltpu.VMEM((_CR, 128), jnp.int32),
            pltpu.VMEM((_CR, 128), jnp.float32),
            pltpu.VMEM((_CR, 128), jnp.float32),
            pltpu.VMEM((_CR, 128), jnp.float32),
            pltpu.SemaphoreType.DMA,
        ],
    )
    return k(user_table, item_table, uid, iid)


def _tc1_body(ui_ref, age_ref, gen_ref, occ_ref, kind_ref, lab_ref,
              aget_ref, gent_ref, occt_ref, kindt_ref,
              s_ref, loss_ref):
    s = ui_ref[...]
    age = age_ref[...]
    for t in range(AGE_NUM):
        s += jnp.where(age == t, aget_ref[t, 0], 0.0)
    gen = gen_ref[...]
    for t in range(GENDER_NUM):
        s += jnp.where(gen == t, gent_ref[t, 0], 0.0)
    occ = occ_ref[...]
    for t in range(OCC_NUM):
        s += jnp.where(occ == t, occt_ref[t, 0], 0.0)
    for kk in range(K):
        kv = kind_ref[kk]
        for t in range(1, KIND_NUM):
            s += jnp.where(kv == t, kindt_ref[t, 0], 0.0)
    s_ref[...] = s
    lab = lab_ref[...]
    p = 1.0 / (1.0 + jnp.exp(-s))
    lossmat = -(lab * jnp.log(p + 1e-6) + (1.0 - lab) * jnp.log(1.0 - p + 1e-6))
    loss_ref[0, 0] = jnp.sum(lossmat) * (1.0 / B)


def _tc1(ui2, age2, gen2, occ2, kindT, lab2, aget, gent, occt, kindt):
    return pl.pallas_call(
        _tc1_body,
        in_specs=[
            pl.BlockSpec(memory_space=pltpu.VMEM),
            pl.BlockSpec(memory_space=pltpu.VMEM),
            pl.BlockSpec(memory_space=pltpu.VMEM),
            pl.BlockSpec(memory_space=pltpu.VMEM),
            pl.BlockSpec(memory_space=pltpu.VMEM),
            pl.BlockSpec(memory_space=pltpu.VMEM),
            pl.BlockSpec(memory_space=pltpu.SMEM),
            pl.BlockSpec(memory_space=pltpu.SMEM),
            pl.BlockSpec(memory_space=pltpu.SMEM),
            pl.BlockSpec(memory_space=pltpu.SMEM),
        ],
        out_specs=[
            pl.BlockSpec(memory_space=pltpu.VMEM),
            pl.BlockSpec(memory_space=pltpu.SMEM),
        ],
        out_shape=[
            jax.ShapeDtypeStruct((128, 128), jnp.float32),
            jax.ShapeDtypeStruct((1, 1), jnp.float32),
        ],
    )(ui2, age2, gen2, occ2, kindT, lab2, aget, gent, occt, kindt)


_JB = 128
_IC = 256
_NSTEP = B // _JB


def _tc2_body(scol_ref, lab_ref, srow_ref, auc_ref, acc_ref):
    step = pl.program_id(0)

    @pl.when(step == 0)
    def _init():
        acc_ref[0] = 0.0
        acc_ref[1] = 0.0

    sj = scol_ref[...]
    labb = lab_ref[...]

    def ic_body(ic, acc):
        si = srow_ref[:, pl.ds(ic * _IC, _IC)]
        return acc + jnp.where(si < sj, 1.0, 0.0)

    acc = lax.fori_loop(0, B // _IC, ic_body,
                        jnp.zeros((_JB, _IC), jnp.float32))
    cnt = jnp.sum(acc, axis=1, keepdims=True)
    acc_ref[0] += jnp.sum(labb * (cnt + 1.0))
    acc_ref[1] += jnp.sum(labb)

    @pl.when(step == _NSTEP - 1)
    def _fin():
        ranksum = acc_ref[0]
        npos = acc_ref[1]
        nneg = B - npos
        auc_ref[0, 0] = (ranksum - npos * (npos + 1.0) / 2.0) / (npos * nneg + 1e-12)


def _tc2(scol, lab, srow):
    return pl.pallas_call(
        _tc2_body,
        grid=(_NSTEP,),
        in_specs=[
            pl.BlockSpec((_JB, 1), lambda i: (i, 0)),
            pl.BlockSpec((_JB, 1), lambda i: (i, 0)),
            pl.BlockSpec((1, B), lambda i: (0, 0)),
        ],
        out_specs=pl.BlockSpec(memory_space=pltpu.SMEM),
        out_shape=jax.ShapeDtypeStruct((1, 1), jnp.float32),
        scratch_shapes=[pltpu.SMEM((2,), jnp.float32)],
    )(scol, lab, srow)


def kernel(userid, itemid, user_age, gender, user_occupation, item_kind, label,
           user_table, item_table, age_table, gender_table, occ_table, kind_table):
    uid = userid.reshape(_NW, _CR, 128)
    iid = itemid.reshape(_NW, _CR, 128)
    ui = _sc_ui_sum(user_table.reshape(-1), item_table.reshape(-1), uid, iid)

    ui2 = ui.reshape(128, 128)
    age2 = user_age.reshape(128, 128)
    gen2 = gender.reshape(128, 128)
    occ2 = user_occupation.reshape(128, 128)
    kindT = item_kind.T.reshape(K, 128, 128)
    lab2 = label.astype(jnp.float32).reshape(128, 128)

    s, loss = _tc1(ui2, age2, gen2, occ2, kindT, lab2,
                   age_table, gender_table, occ_table, kind_table)

    scol = s.reshape(B, 1)
    srow = s.reshape(1, B)
    labcol = label.astype(jnp.float32).reshape(B, 1)
    auc = _tc2(scol, labcol, srow)

    return (loss.reshape(()), auc.reshape(()))


# speedup vs baseline: 6.6793x; 6.6793x over previous
import functools

import jax
import jax.numpy as jnp
from jax import lax
from jax.experimental import pallas as pl
from jax.experimental.pallas import tpu as pltpu
from jax.experimental.pallas import tpu_sc as plsc

B = 16384
K = 20
AGE_NUM = 8
GENDER_NUM = 3
OCC_NUM = 22
KIND_NUM = 19


_NC, _NS = 2, 16
_NW = _NC * _NS
_CH = B // _NW
_CR = _CH // 128


def _sc_body(ut_hbm, it_hbm, uid_hbm, iid_hbm, out_hbm,
             uidx_v, iidx_v, urow_v, irow_v, sum_v, sem):
    wid = lax.axis_index("s") * _NC + lax.axis_index("c")
    pltpu.sync_copy(uid_hbm.at[wid], uidx_v)
    pltpu.sync_copy(iid_hbm.at[wid], iidx_v)
    cps = []
    for j in range(_CR):
        cps.append(pltpu.async_copy(ut_hbm.at[uidx_v.at[j]], urow_v.at[j], sem))
        cps.append(pltpu.async_copy(it_hbm.at[iidx_v.at[j]], irow_v.at[j], sem))
    for cp in cps:
        cp.wait()
    for j in range(_CR):
        for i in range(8):
            s = pl.ds(i * 16, 16)
            sum_v[j, s] = urow_v[j, s] + irow_v[j, s]
    pltpu.sync_copy(sum_v, out_hbm.at[wid])


def _sc_ui_sum(user_table, item_table, uid, iid):
    mesh = plsc.VectorSubcoreMesh(core_axis_name="c", subcore_axis_name="s")
    k = pl.kernel(
        _sc_body,
        out_type=jax.ShapeDtypeStruct((_NW, _CR, 128), jnp.float32),
        mesh=mesh,
        scratch_types=[
            pltpu.VMEM((_CR, 128), jnp.int32),
            pltpu.VMEM((_CR, 128), jnp.int32),
            pltpu.VMEM((_CR, 128), jnp.float32),
            pltpu.VMEM((_CR, 128), jnp.float32),
            pltpu.VMEM((_CR, 128), jnp.float32),
            pltpu.SemaphoreType.DMA,
        ],
    )
    return k(user_table, item_table, uid, iid)


def _tc1_body(ui_ref, age_ref, gen_ref, occ_ref, kind_ref, lab_ref,
              aget_ref, gent_ref, occt_ref, kindt_ref,
              bid_ref, loss_ref):
    s = ui_ref[...]
    age = age_ref[...]
    for t in range(AGE_NUM):
        s += jnp.where(age == t, aget_ref[t, 0], 0.0)
    gen = gen_ref[...]
    for t in range(GENDER_NUM):
        s += jnp.where(gen == t, gent_ref[t, 0], 0.0)
    occ = occ_ref[...]
    for t in range(OCC_NUM):
        s += jnp.where(occ == t, occt_ref[t, 0], 0.0)
    for kk in range(K):
        kv = kind_ref[kk]
        for t in range(1, KIND_NUM):
            s += jnp.where(kv == t, kindt_ref[t, 0], 0.0)
    lab = lab_ref[...]
    p = 1.0 / (1.0 + jnp.exp(-s))
    lossmat = -(lab * jnp.log(p + 1e-6) + (1.0 - lab) * jnp.log(1.0 - p + 1e-6))
    loss_ref[0, 0] = jnp.sum(lossmat) * (1.0 / B)
    smin = jnp.min(s)
    smax = jnp.max(s)
    u = (s - smin) / (smax - smin + 1e-30)
    bid = jnp.clip(jnp.floor(u * _NB).astype(jnp.int32), 0, _NB - 1)
    bid_ref[...] = bid


def _tc1(ui2, age2, gen2, occ2, kindT, lab2, aget, gent, occt, kindt):
    return pl.pallas_call(
        _tc1_body,
        in_specs=[
            pl.BlockSpec(memory_space=pltpu.VMEM),
            pl.BlockSpec(memory_space=pltpu.VMEM),
            pl.BlockSpec(memory_space=pltpu.VMEM),
            pl.BlockSpec(memory_space=pltpu.VMEM),
            pl.BlockSpec(memory_space=pltpu.VMEM),
            pl.BlockSpec(memory_space=pltpu.VMEM),
            pl.BlockSpec(memory_space=pltpu.SMEM),
            pl.BlockSpec(memory_space=pltpu.SMEM),
            pl.BlockSpec(memory_space=pltpu.SMEM),
            pl.BlockSpec(memory_space=pltpu.SMEM),
        ],
        out_specs=[
            pl.BlockSpec(memory_space=pltpu.VMEM),
            pl.BlockSpec(memory_space=pltpu.SMEM),
        ],
        out_shape=[
            jax.ShapeDtypeStruct((128, 128), jnp.int32),
            jax.ShapeDtypeStruct((1, 1), jnp.float32),
        ],
    )(ui2, age2, gen2, occ2, kindT, lab2, aget, gent, occt, kindt)


_NB = 1024
_EB = 1024
_NBLK = B // _EB


def _tc2_body(bid_ref, lab_ref, auc_ref, hcol_ref, vcol_ref, acc_ref):
    step = pl.program_id(0)

    @pl.when(step == 0)
    def _init():
        hcol_ref[...] = jnp.zeros((_NB, 1), jnp.float32)
        acc_ref[0] = 0.0
        acc_ref[1] = 0.0

    bid = bid_ref[...]
    nbio = lax.broadcasted_iota(jnp.int32, (_EB, _NB), 1)
    oh = (bid == nbio).astype(jnp.float32)

    @pl.when(step < _NBLK)
    def _pass1():
        ones = jnp.ones((_EB, 1), jnp.float32)
        hcol_ref[...] += lax.dot_general(
            oh, ones, (((0,), (0,)), ((), ())),
            preferred_element_type=jnp.float32)

    @pl.when(step == _NBLK)
    def _mkcum():
        rowio = lax.broadcasted_iota(jnp.int32, (_NB, _NB), 0)
        colio = lax.broadcasted_iota(jnp.int32, (_NB, _NB), 1)
        c = (colio < rowio).astype(jnp.float32)
        h = hcol_ref[...]
        excl = jnp.dot(c, h, preferred_element_type=jnp.float32)
        vcol_ref[...] = excl + 0.5 * h

    @pl.when(step >= _NBLK)
    def _pass2():
        rank = jnp.dot(oh, vcol_ref[...],
                       preferred_element_type=jnp.float32)
        labb = lab_ref[...]
        acc_ref[0] += jnp.sum(labb * (rank + 0.5))
        acc_ref[1] += jnp.sum(labb)

    @pl.when(step == 2 * _NBLK - 1)
    def _fin():
        ranksum = acc_ref[0]
        npos = acc_ref[1]
        nneg = B - npos
        auc_ref[0, 0] = (ranksum - npos * (npos + 1.0) / 2.0) / (npos * nneg + 1e-12)


def _tc2(bidcol, labcol):
    return pl.pallas_call(
        _tc2_body,
        grid=(2 * _NBLK,),
        in_specs=[
            pl.BlockSpec((_EB, 1), lambda i: (i % _NBLK, 0)),
            pl.BlockSpec((_EB, 1), lambda i: (i % _NBLK, 0)),
        ],
        out_specs=pl.BlockSpec(memory_space=pltpu.SMEM),
        out_shape=jax.ShapeDtypeStruct((1, 1), jnp.float32),
        scratch_shapes=[
            pltpu.VMEM((_NB, 1), jnp.float32),
            pltpu.VMEM((_NB, 1), jnp.float32),
            pltpu.SMEM((2,), jnp.float32),
        ],
    )(bidcol, labcol)


def kernel(userid, itemid, user_age, gender, user_occupation, item_kind, label,
           user_table, item_table, age_table, gender_table, occ_table, kind_table):
    uid = userid.reshape(_NW, _CR, 128)
    iid = itemid.reshape(_NW, _CR, 128)
    ui = _sc_ui_sum(user_table.reshape(-1), item_table.reshape(-1), uid, iid)

    ui2 = ui.reshape(128, 128)
    age2 = user_age.reshape(128, 128)
    gen2 = gender.reshape(128, 128)
    occ2 = user_occupation.reshape(128, 128)
    kindT = item_kind.T.reshape(K, 128, 128)
    lab2 = label.astype(jnp.float32).reshape(128, 128)

    bid, loss = _tc1(ui2, age2, gen2, occ2, kindT, lab2,
                     age_table, gender_table, occ_table, kind_table)

    bidcol = bid.reshape(B, 1)
    labcol = label.astype(jnp.float32).reshape(B, 1)
    auc = _tc2(bidcol, labcol)

    return (loss.reshape(()), auc.reshape(()))
